# Initial kernel scaffold; baseline (speedup 1.0000x reference)
import jax
import jax.numpy as jnp
from jax.experimental import pallas as pl


def kernel(x_gene, x_disease, params, edge_index_g2d, edge_index_d2g, edge_label_index_g2d, edge_label_index_d2g):
    raise NotImplementedError("write your pallas kernel here")


# speedup vs baseline: 4.3969x; 4.3969x over previous
import functools

import jax
import jax.numpy as jnp
from jax import lax
from jax.experimental import pallas as pl
from jax.experimental.pallas import tpu as pltpu
from jax.experimental.pallas import tpu_sc as plsc

N = 10000
D = 128
E = 320000
L = 65536

E_ROWS = E // 128
L_ROWS = L // 128
NS = 16
STRIPE = 624


def _mesh():
    return plsc.VectorSubcoreMesh(core_axis_name="c", subcore_axis_name="s")


def _agg_body(x_gene, x_dis, src_g2d, dst_g2d, src_d2g, dst_d2g,
              sum_dis, sum_gene,
              idx_src_v, idx_dst_v, rows_v, sum_acc, sem):
    c = lax.axis_index("c")
    s = lax.axis_index("s")

    def fill(i, carry):
        for j in range(8):
            rows_v[i, pl.ds(j * 16, 16)] = jnp.zeros((16,), jnp.float32)
        return carry
    lax.fori_loop(0, 128, fill, 0)

    base = s * STRIPE
    off = 0
    for sz in (128, 128, 128, 128, 112):
        pltpu.sync_copy(rows_v.at[pl.ds(0, sz)], sum_acc.at[pl.ds(base + off, sz)])
        off += sz

    @pl.when(s == NS - 1)
    def _():
        pltpu.sync_copy(rows_v.at[pl.ds(0, 16)], sum_acc.at[pl.ds(NS * STRIPE, 16)])
    plsc.subcore_barrier()

    def run_type(x_ref, src_ref, dst_ref, sum_out):
        def body(i, carry):
            r = s + i * NS

            @pl.when(r < E_ROWS)
            def _():
                pltpu.sync_copy(src_ref.at[pl.ds(r * 128, 128)], idx_src_v)
                pltpu.sync_copy(dst_ref.at[pl.ds(r * 128, 128)], idx_dst_v)
                pltpu.async_copy(x_ref.at[idx_src_v], rows_v, sem).wait()
                pltpu.sync_copy(rows_v, sum_acc.at[idx_dst_v], add=True)
            return carry
        lax.fori_loop(0, (E_ROWS + NS - 1) // NS, body, 0)
        plsc.subcore_barrier()
        pltpu.sync_copy(sum_acc.at[pl.ds(base, STRIPE)],
                        sum_out.at[pl.ds(base, STRIPE)])

        @pl.when(s == NS - 1)
        def _():
            pltpu.sync_copy(sum_acc.at[pl.ds(NS * STRIPE, 16)],
                            sum_out.at[pl.ds(NS * STRIPE, 16)])

    @pl.when(c == 0)
    def _():
        run_type(x_gene, src_g2d, dst_g2d, sum_dis)

    @pl.when(c == 1)
    def _():
        run_type(x_dis, src_d2g, dst_d2g, sum_gene)


def _make_agg():
    return pl.kernel(
        _agg_body,
        out_type=[
            jax.ShapeDtypeStruct((N, D), jnp.float32),
            jax.ShapeDtypeStruct((N, D), jnp.float32),
        ],
        mesh=_mesh(),
        scratch_types=[
            pltpu.VMEM((128,), jnp.int32),
            pltpu.VMEM((128,), jnp.int32),
            pltpu.VMEM((128, D), jnp.float32),
            pltpu.VMEM_SHARED((N, D), jnp.float32),
            pltpu.SemaphoreType.DMA,
        ],
    )


def _cnt_body(dst_g2d, dst_d2g, cnt_dis, cnt_gene,
              idx_dst_v, zcnt_v, ones_v, cnt_acc, sem):
    c = lax.axis_index("c")
    s = lax.axis_index("s")

    def fill(i, carry):
        for j in range(8):
            zcnt_v[i, pl.ds(j * 16, 16)] = jnp.zeros((16,), jnp.float32)
            ones_v[i, pl.ds(j * 16, 16)] = jnp.ones((16,), jnp.float32)
        return carry
    lax.fori_loop(0, 128, fill, 0)

    base = s * STRIPE
    off = 0
    for sz in (128, 128, 128, 128, 112):
        pltpu.sync_copy(zcnt_v.at[pl.ds(0, sz)], cnt_acc.at[pl.ds(base + off, sz)])
        off += sz

    @pl.when(s == NS - 1)
    def _():
        pltpu.sync_copy(zcnt_v.at[pl.ds(0, 16)], cnt_acc.at[pl.ds(NS * STRIPE, 16)])
    plsc.subcore_barrier()

    def run_type(dst_ref, cnt_out):
        def body(i, carry):
            r = s + i * NS

            @pl.when(r < E_ROWS)
            def _():
                pltpu.sync_copy(dst_ref.at[pl.ds(r * 128, 128)], idx_dst_v)
                pltpu.sync_copy(ones_v, cnt_acc.at[idx_dst_v], add=True)
            return carry
        lax.fori_loop(0, (E_ROWS + NS - 1) // NS, body, 0)
        plsc.subcore_barrier()
        pltpu.sync_copy(cnt_acc.at[pl.ds(base, STRIPE)],
                        cnt_out.at[pl.ds(base, STRIPE)])

        @pl.when(s == NS - 1)
        def _():
            pltpu.sync_copy(cnt_acc.at[pl.ds(NS * STRIPE, 16)],
                            cnt_out.at[pl.ds(NS * STRIPE, 16)])

    @pl.when(c == 0)
    def _():
        run_type(dst_g2d, cnt_dis)

    @pl.when(c == 1)
    def _():
        run_type(dst_d2g, cnt_gene)


def _make_cnt():
    return pl.kernel(
        _cnt_body,
        out_type=[
            jax.ShapeDtypeStruct((N, 128), jnp.float32),
            jax.ShapeDtypeStruct((N, 128), jnp.float32),
        ],
        mesh=_mesh(),
        scratch_types=[
            pltpu.VMEM((128,), jnp.int32),
            pltpu.VMEM((128, 128), jnp.float32),
            pltpu.VMEM((128, 128), jnp.float32),
            pltpu.VMEM_SHARED((N, 128), jnp.float32),
            pltpu.SemaphoreType.DMA,
        ],
    )


def _dense_one(x_dst, ssum, cnt, Wd, bd, Ws, bs, Wu, bu, gam, bet, relu):
    cnt1 = cnt[:, 0:1]
    aggr = jnp.where(cnt1 > 0, ssum / jnp.maximum(cnt1, 1.0), 0.0)
    dst_msg = jnp.dot(x_dst, Wd, preferred_element_type=jnp.float32) + bd
    src_msg = jnp.dot(aggr, Ws, preferred_element_type=jnp.float32) + bs
    h = (jnp.dot(dst_msg, Wu[0:D, :], preferred_element_type=jnp.float32)
         + jnp.dot(src_msg, Wu[D:2 * D, :], preferred_element_type=jnp.float32)
         + bu)
    m = jnp.mean(h, axis=0, keepdims=True)
    hc = h - m
    v = jnp.mean(hc * hc, axis=0, keepdims=True)
    h = gam * hc / jnp.sqrt(v + 1e-5) + bet
    if relu:
        h = jnp.where(h >= 0, h, 0.01 * h)
    return h


def _dense_body(x_ref, s_ref, c_ref,
                wd, bd, ws, bs, wu, bu, gam, bet,
                h_out, *, relu):
    h_out[...] = _dense_one(x_ref[...], s_ref[...], c_ref[...],
                            wd[...], bd[...], ws[...], bs[...],
                            wu[...], bu[...], gam[...], bet[...], relu)


def _make_dense(relu):
    return pl.pallas_call(
        functools.partial(_dense_body, relu=relu),
        out_shape=jax.ShapeDtypeStruct((N, D), jnp.float32),
    )


def _dec_body(hg, hd, a_g2d, b_g2d, a_d2g, b_d2g,
              part_g2d, part_d2g,
              ia_v, ib_v, ra_v, rb_v, o_v, sem):
    c = lax.axis_index("c")
    s = lax.axis_index("s")

    def run(tabA, tabB, a_ref, b_ref, out_ref):
        def chunk(i, carry):
            r = s * (L_ROWS // NS) + i
            pltpu.sync_copy(a_ref.at[pl.ds(r * 128, 128)], ia_v)
            pltpu.sync_copy(b_ref.at[pl.ds(r * 128, 128)], ib_v)
            cpA = pltpu.async_copy(tabA.at[ia_v], ra_v, sem)
            cpB = pltpu.async_copy(tabB.at[ib_v], rb_v, sem)
            cpA.wait()
            cpB.wait()

            def pair(p, carry2):
                acc = ra_v[p, pl.ds(0, 16)] * rb_v[p, pl.ds(0, 16)]
                for j in range(1, 8):
                    acc = acc + ra_v[p, pl.ds(j * 16, 16)] * rb_v[p, pl.ds(j * 16, 16)]
                o_v[p, :] = acc
                return carry2
            lax.fori_loop(0, 128, pair, 0, unroll=2)
            pltpu.sync_copy(o_v, out_ref.at[pl.ds(r * 128, 128)])
            return carry
        lax.fori_loop(0, L_ROWS // NS, chunk, 0)

    @pl.when(c == 0)
    def _():
        run(hg, hd, a_g2d, b_g2d, part_g2d)

    @pl.when(c == 1)
    def _():
        run(hd, hg, a_d2g, b_d2g, part_d2g)


def _make_dec():
    return pl.kernel(
        _dec_body,
        out_type=[
            jax.ShapeDtypeStruct((L, 16), jnp.float32),
            jax.ShapeDtypeStruct((L, 16), jnp.float32),
        ],
        mesh=_mesh(),
        scratch_types=[
            pltpu.VMEM((128,), jnp.int32),
            pltpu.VMEM((128,), jnp.int32),
            pltpu.VMEM((128, D), jnp.float32),
            pltpu.VMEM((128, D), jnp.float32),
            pltpu.VMEM((128, 16), jnp.float32),
            pltpu.SemaphoreType.DMA,
        ],
    )


def _finish_body(pg_ref, pd_ref, o0_ref, o1_ref):
    grp = lax.broadcasted_iota(jnp.int32, (16 * 128, 128), 0) // 16
    col = lax.broadcasted_iota(jnp.int32, (16 * 128, 128), 1)
    m = (grp == col).astype(jnp.float32)
    hi = jax.lax.Precision.HIGHEST
    o0_ref[...] = jnp.dot(pg_ref[...], m, precision=hi,
                          preferred_element_type=jnp.float32)
    o1_ref[...] = jnp.dot(pd_ref[...], m, precision=hi,
                          preferred_element_type=jnp.float32)


def _make_finish():
    return pl.pallas_call(
        _finish_body,
        out_shape=[
            jax.ShapeDtypeStruct((L_ROWS, 128), jnp.float32),
            jax.ShapeDtypeStruct((L_ROWS, 128), jnp.float32),
        ],
    )


def kernel(x_gene, x_disease, params, edge_index_g2d, edge_index_d2g,
           edge_label_index_g2d, edge_label_index_d2g):
    src_g2d = edge_index_g2d[0].astype(jnp.int32)
    dst_g2d = edge_index_g2d[1].astype(jnp.int32)
    src_d2g = edge_index_d2g[0].astype(jnp.int32)
    dst_d2g = edge_index_d2g[1].astype(jnp.int32)
    a_g2d = edge_label_index_g2d[0].astype(jnp.int32)
    b_g2d = edge_label_index_g2d[1].astype(jnp.int32)
    a_d2g = edge_label_index_d2g[0].astype(jnp.int32)
    b_d2g = edge_label_index_d2g[1].astype(jnp.int32)

    def wset(p):
        return (p['W_dst'], p['b_dst'].reshape(1, D), p['W_src'],
                p['b_src'].reshape(1, D), p['W_upd'], p['b_upd'].reshape(1, D))

    def bnset(p):
        return (p['gamma'].reshape(1, D), p['beta'].reshape(1, D))

    agg = _make_agg()
    dec = _make_dec()

    cnt_d, cnt_g = _make_cnt()(dst_g2d, dst_d2g)

    dense_relu = _make_dense(True)
    dense_lin = _make_dense(False)

    sum_d1, sum_g1 = agg(
        x_gene, x_disease, src_g2d, dst_g2d, src_d2g, dst_d2g)
    h1d = dense_relu(x_disease, sum_d1, cnt_d,
                     *wset(params['conv1_g2d']), *bnset(params['bn1_disease']))
    h1g = dense_relu(x_gene, sum_g1, cnt_g,
                     *wset(params['conv1_d2g']), *bnset(params['bn1_gene']))

    sum_d2, sum_g2 = agg(
        h1g, h1d, src_g2d, dst_g2d, src_d2g, dst_d2g)
    h2d = dense_lin(h1d, sum_d2, cnt_d,
                    *wset(params['conv2_g2d']), *bnset(params['bn2_disease']))
    h2g = dense_lin(h1g, sum_g2, cnt_g,
                    *wset(params['conv2_d2g']), *bnset(params['bn2_gene']))

    pg, pd = dec(h2g, h2d, a_g2d, b_g2d, a_d2g, b_d2g)
    o0, o1 = _make_finish()(pg.reshape(L_ROWS, 16 * 128), pd.reshape(L_ROWS, 16 * 128))
    return jnp.stack([o0.reshape(L), o1.reshape(L)], axis=0)


# speedup vs baseline: 6.2441x; 1.4201x over previous
import functools

import jax
import jax.numpy as jnp
from jax import lax
from jax.experimental import pallas as pl
from jax.experimental.pallas import tpu as pltpu
from jax.experimental.pallas import tpu_sc as plsc

N = 10000
D = 128
E = 320000
L = 65536

E_ROWS = E // 128
L_ROWS = L // 128
NS = 16
STRIPE = 624


def _mesh():
    return plsc.VectorSubcoreMesh(core_axis_name="c", subcore_axis_name="s")


GRP = 8
N_GROUPS = E_ROWS // GRP
REM = E_ROWS - N_GROUPS * GRP


def _agg_body(x_gene, x_dis, src_g2d, dst_g2d, src_d2g, dst_d2g,
              sum_dis, sum_gene,
              is8, id8, rows_a, rows_b, sum_acc, sem_a, sem_b):
    c = lax.axis_index("c")
    s = lax.axis_index("s")
    rows = (rows_a, rows_b)
    sems = (sem_a, sem_b)

    def fill(i, carry):
        for j in range(8):
            rows_a[i, pl.ds(j * 16, 16)] = jnp.zeros((16,), jnp.float32)
        return carry
    lax.fori_loop(0, 128, fill, 0)

    base = s * STRIPE
    off = 0
    for sz in (128, 128, 128, 128, 112):
        pltpu.sync_copy(rows_a.at[pl.ds(0, sz)], sum_acc.at[pl.ds(base + off, sz)])
        off += sz

    @pl.when(s == NS - 1)
    def _():
        pltpu.sync_copy(rows_a.at[pl.ds(0, 16)], sum_acc.at[pl.ds(NS * STRIPE, 16)])
    plsc.subcore_barrier()

    def run_type(x_ref, src_ref, dst_ref, sum_out):
        def body(i, carry):
            g = s + i * NS

            @pl.when(g < N_GROUPS)
            def _():
                pltpu.sync_copy(src_ref.at[pl.ds(g * GRP, GRP)], is8)
                pltpu.sync_copy(dst_ref.at[pl.ds(g * GRP, GRP)], id8)
                cp = pltpu.async_copy(x_ref.at[is8.at[0]], rows[0], sems[0])
                for j in range(GRP):
                    nxt = (j + 1) % 2
                    if j + 1 < GRP:
                        cp_n = pltpu.async_copy(x_ref.at[is8.at[j + 1]],
                                                rows[nxt], sems[nxt])
                    cp.wait()
                    pltpu.sync_copy(rows[j % 2], sum_acc.at[id8.at[j]], add=True)
                    if j + 1 < GRP:
                        cp = cp_n
            return carry
        lax.fori_loop(0, (N_GROUPS + NS - 1) // NS, body, 0)

        @pl.when(s == 0)
        def _():
            pltpu.sync_copy(src_ref.at[pl.ds(N_GROUPS * GRP, REM)],
                            is8.at[pl.ds(0, REM)])
            pltpu.sync_copy(dst_ref.at[pl.ds(N_GROUPS * GRP, REM)],
                            id8.at[pl.ds(0, REM)])
            for j in range(REM):
                pltpu.async_copy(x_ref.at[is8.at[j]], rows_a, sem_a).wait()
                pltpu.sync_copy(rows_a, sum_acc.at[id8.at[j]], add=True)

        plsc.subcore_barrier()
        pltpu.sync_copy(sum_acc.at[pl.ds(base, STRIPE)],
                        sum_out.at[pl.ds(base, STRIPE)])

        @pl.when(s == NS - 1)
        def _():
            pltpu.sync_copy(sum_acc.at[pl.ds(NS * STRIPE, 16)],
                            sum_out.at[pl.ds(NS * STRIPE, 16)])

    @pl.when(c == 0)
    def _():
        run_type(x_gene, src_g2d, dst_g2d, sum_dis)

    @pl.when(c == 1)
    def _():
        run_type(x_dis, src_d2g, dst_d2g, sum_gene)


def _make_agg():
    return pl.kernel(
        _agg_body,
        out_type=[
            jax.ShapeDtypeStruct((N, D), jnp.float32),
            jax.ShapeDtypeStruct((N, D), jnp.float32),
        ],
        mesh=_mesh(),
        scratch_types=[
            pltpu.VMEM((GRP, 128), jnp.int32),
            pltpu.VMEM((GRP, 128), jnp.int32),
            pltpu.VMEM((128, D), jnp.float32),
            pltpu.VMEM((128, D), jnp.float32),
            pltpu.VMEM_SHARED((N, D), jnp.float32),
            pltpu.SemaphoreType.DMA,
            pltpu.SemaphoreType.DMA,
        ],
    )


def _cnt_body(dst_g2d, dst_d2g, cnt_dis, cnt_gene,
              idx_dst_v, zcnt_v, ones_v, cnt_acc, sem):
    c = lax.axis_index("c")
    s = lax.axis_index("s")

    def fill(i, carry):
        for j in range(8):
            zcnt_v[i, pl.ds(j * 16, 16)] = jnp.zeros((16,), jnp.float32)
            ones_v[i, pl.ds(j * 16, 16)] = jnp.ones((16,), jnp.float32)
        return carry
    lax.fori_loop(0, 128, fill, 0)

    base = s * STRIPE
    off = 0
    for sz in (128, 128, 128, 128, 112):
        pltpu.sync_copy(zcnt_v.at[pl.ds(0, sz)], cnt_acc.at[pl.ds(base + off, sz)])
        off += sz

    @pl.when(s == NS - 1)
    def _():
        pltpu.sync_copy(zcnt_v.at[pl.ds(0, 16)], cnt_acc.at[pl.ds(NS * STRIPE, 16)])
    plsc.subcore_barrier()

    def run_type(dst_ref, cnt_out):
        def body(i, carry):
            r = s + i * NS

            @pl.when(r < E_ROWS)
            def _():
                pltpu.sync_copy(dst_ref.at[pl.ds(r * 128, 128)], idx_dst_v)
                pltpu.sync_copy(ones_v, cnt_acc.at[idx_dst_v], add=True)
            return carry
        lax.fori_loop(0, (E_ROWS + NS - 1) // NS, body, 0)
        plsc.subcore_barrier()
        pltpu.sync_copy(cnt_acc.at[pl.ds(base, STRIPE)],
                        cnt_out.at[pl.ds(base, STRIPE)])

        @pl.when(s == NS - 1)
        def _():
            pltpu.sync_copy(cnt_acc.at[pl.ds(NS * STRIPE, 16)],
                            cnt_out.at[pl.ds(NS * STRIPE, 16)])

    @pl.when(c == 0)
    def _():
        run_type(dst_g2d, cnt_dis)

    @pl.when(c == 1)
    def _():
        run_type(dst_d2g, cnt_gene)


def _make_cnt():
    return pl.kernel(
        _cnt_body,
        out_type=[
            jax.ShapeDtypeStruct((N, 128), jnp.float32),
            jax.ShapeDtypeStruct((N, 128), jnp.float32),
        ],
        mesh=_mesh(),
        scratch_types=[
            pltpu.VMEM((128,), jnp.int32),
            pltpu.VMEM((128, 128), jnp.float32),
            pltpu.VMEM((128, 128), jnp.float32),
            pltpu.VMEM_SHARED((N, 128), jnp.float32),
            pltpu.SemaphoreType.DMA,
        ],
    )


def _dense_one(x_dst, ssum, cnt, Wd, bd, Ws, bs, Wu, bu, gam, bet, relu):
    cnt1 = cnt[:, 0:1]
    aggr = jnp.where(cnt1 > 0, ssum / jnp.maximum(cnt1, 1.0), 0.0)
    dst_msg = jnp.dot(x_dst, Wd, preferred_element_type=jnp.float32) + bd
    src_msg = jnp.dot(aggr, Ws, preferred_element_type=jnp.float32) + bs
    h = (jnp.dot(dst_msg, Wu[0:D, :], preferred_element_type=jnp.float32)
         + jnp.dot(src_msg, Wu[D:2 * D, :], preferred_element_type=jnp.float32)
         + bu)
    m = jnp.mean(h, axis=0, keepdims=True)
    hc = h - m
    v = jnp.mean(hc * hc, axis=0, keepdims=True)
    h = gam * hc / jnp.sqrt(v + 1e-5) + bet
    if relu:
        h = jnp.where(h >= 0, h, 0.01 * h)
    return h


def _dense_body(x_ref, s_ref, c_ref,
                wd, bd, ws, bs, wu, bu, gam, bet,
                h_out, *, relu):
    h_out[...] = _dense_one(x_ref[...], s_ref[...], c_ref[...],
                            wd[...], bd[...], ws[...], bs[...],
                            wu[...], bu[...], gam[...], bet[...], relu)


def _make_dense(relu):
    return pl.pallas_call(
        functools.partial(_dense_body, relu=relu),
        out_shape=jax.ShapeDtypeStruct((N, D), jnp.float32),
    )


def _dec_body(hg, hd, a_g2d, b_g2d, a_d2g, b_d2g,
              part_g2d, part_d2g,
              ia_v, ib_v, ra_v, rb_v, o_v, sem):
    c = lax.axis_index("c")
    s = lax.axis_index("s")

    def run(tabA, tabB, a_ref, b_ref, out_ref):
        def chunk(i, carry):
            r = s * (L_ROWS // NS) + i
            pltpu.sync_copy(a_ref.at[pl.ds(r * 128, 128)], ia_v)
            pltpu.sync_copy(b_ref.at[pl.ds(r * 128, 128)], ib_v)
            cpA = pltpu.async_copy(tabA.at[ia_v], ra_v, sem)
            cpB = pltpu.async_copy(tabB.at[ib_v], rb_v, sem)
            cpA.wait()
            cpB.wait()

            def pair(p, carry2):
                acc = ra_v[p, pl.ds(0, 16)] * rb_v[p, pl.ds(0, 16)]
                for j in range(1, 8):
                    acc = acc + ra_v[p, pl.ds(j * 16, 16)] * rb_v[p, pl.ds(j * 16, 16)]
                o_v[p, :] = acc
                return carry2
            lax.fori_loop(0, 128, pair, 0, unroll=2)
            pltpu.sync_copy(o_v, out_ref.at[pl.ds(r * 128, 128)])
            return carry
        lax.fori_loop(0, L_ROWS // NS, chunk, 0)

    @pl.when(c == 0)
    def _():
        run(hg, hd, a_g2d, b_g2d, part_g2d)

    @pl.when(c == 1)
    def _():
        run(hd, hg, a_d2g, b_d2g, part_d2g)


def _make_dec():
    return pl.kernel(
        _dec_body,
        out_type=[
            jax.ShapeDtypeStruct((L, 16), jnp.float32),
            jax.ShapeDtypeStruct((L, 16), jnp.float32),
        ],
        mesh=_mesh(),
        scratch_types=[
            pltpu.VMEM((128,), jnp.int32),
            pltpu.VMEM((128,), jnp.int32),
            pltpu.VMEM((128, D), jnp.float32),
            pltpu.VMEM((128, D), jnp.float32),
            pltpu.VMEM((128, 16), jnp.float32),
            pltpu.SemaphoreType.DMA,
        ],
    )


def _finish_body(pg_ref, pd_ref, o0_ref, o1_ref):
    grp = lax.broadcasted_iota(jnp.int32, (16 * 128, 128), 0) // 16
    col = lax.broadcasted_iota(jnp.int32, (16 * 128, 128), 1)
    m = (grp == col).astype(jnp.float32)
    hi = jax.lax.Precision.HIGHEST
    o0_ref[...] = jnp.dot(pg_ref[...], m, precision=hi,
                          preferred_element_type=jnp.float32)
    o1_ref[...] = jnp.dot(pd_ref[...], m, precision=hi,
                          preferred_element_type=jnp.float32)


def _make_finish():
    return pl.pallas_call(
        _finish_body,
        out_shape=[
            jax.ShapeDtypeStruct((L_ROWS, 128), jnp.float32),
            jax.ShapeDtypeStruct((L_ROWS, 128), jnp.float32),
        ],
    )


def kernel(x_gene, x_disease, params, edge_index_g2d, edge_index_d2g,
           edge_label_index_g2d, edge_label_index_d2g):
    src_g2d = edge_index_g2d[0].astype(jnp.int32)
    dst_g2d = edge_index_g2d[1].astype(jnp.int32)
    src_d2g = edge_index_d2g[0].astype(jnp.int32)
    dst_d2g = edge_index_d2g[1].astype(jnp.int32)
    a_g2d = edge_label_index_g2d[0].astype(jnp.int32)
    b_g2d = edge_label_index_g2d[1].astype(jnp.int32)
    a_d2g = edge_label_index_d2g[0].astype(jnp.int32)
    b_d2g = edge_label_index_d2g[1].astype(jnp.int32)

    def wset(p):
        return (p['W_dst'], p['b_dst'].reshape(1, D), p['W_src'],
                p['b_src'].reshape(1, D), p['W_upd'], p['b_upd'].reshape(1, D))

    def bnset(p):
        return (p['gamma'].reshape(1, D), p['beta'].reshape(1, D))

    agg = _make_agg()
    dec = _make_dec()

    cnt_d, cnt_g = _make_cnt()(dst_g2d, dst_d2g)

    dense_relu = _make_dense(True)
    dense_lin = _make_dense(False)

    src_g2d_2 = src_g2d.reshape(E_ROWS, 128)
    dst_g2d_2 = dst_g2d.reshape(E_ROWS, 128)
    src_d2g_2 = src_d2g.reshape(E_ROWS, 128)
    dst_d2g_2 = dst_d2g.reshape(E_ROWS, 128)

    sum_d1, sum_g1 = agg(
        x_gene, x_disease, src_g2d_2, dst_g2d_2, src_d2g_2, dst_d2g_2)
    h1d = dense_relu(x_disease, sum_d1, cnt_d,
                     *wset(params['conv1_g2d']), *bnset(params['bn1_disease']))
    h1g = dense_relu(x_gene, sum_g1, cnt_g,
                     *wset(params['conv1_d2g']), *bnset(params['bn1_gene']))

    sum_d2, sum_g2 = agg(
        h1g, h1d, src_g2d_2, dst_g2d_2, src_d2g_2, dst_d2g_2)
    h2d = dense_lin(h1d, sum_d2, cnt_d,
                    *wset(params['conv2_g2d']), *bnset(params['bn2_disease']))
    h2g = dense_lin(h1g, sum_g2, cnt_g,
                    *wset(params['conv2_d2g']), *bnset(params['bn2_gene']))

    pg, pd = dec(h2g, h2d, a_g2d, b_g2d, a_d2g, b_d2g)
    o0, o1 = _make_finish()(pg.reshape(L_ROWS, 16 * 128), pd.reshape(L_ROWS, 16 * 128))
    return jnp.stack([o0.reshape(L), o1.reshape(L)], axis=0)


# speedup vs baseline: 7.1575x; 1.1463x over previous
import functools

import jax
import jax.numpy as jnp
from jax import lax
from jax.experimental import pallas as pl
from jax.experimental.pallas import tpu as pltpu
from jax.experimental.pallas import tpu_sc as plsc

N = 10000
D = 128
E = 320000
L = 65536

E_ROWS = E // 128
L_ROWS = L // 128
NS = 16
STRIPE = 624


def _mesh():
    return plsc.VectorSubcoreMesh(core_axis_name="c", subcore_axis_name="s")


GRP = 8
N_GROUPS = E_ROWS // GRP
REM = E_ROWS - N_GROUPS * GRP


def _agg_body(x_gene, x_dis, src_g2d, dst_g2d, src_d2g, dst_d2g,
              sum_dis, sum_gene,
              is8, id8, rows_a, rows_b, sum_acc, sem_a, sem_b):
    c = lax.axis_index("c")
    s = lax.axis_index("s")
    rows = (rows_a, rows_b)
    sems = (sem_a, sem_b)

    def fill(i, carry):
        for j in range(8):
            rows_a[i, pl.ds(j * 16, 16)] = jnp.zeros((16,), jnp.float32)
        return carry
    lax.fori_loop(0, 128, fill, 0)

    base = s * STRIPE
    off = 0
    for sz in (128, 128, 128, 128, 112):
        pltpu.sync_copy(rows_a.at[pl.ds(0, sz)], sum_acc.at[pl.ds(base + off, sz)])
        off += sz

    @pl.when(s == NS - 1)
    def _():
        pltpu.sync_copy(rows_a.at[pl.ds(0, 16)], sum_acc.at[pl.ds(NS * STRIPE, 16)])
    plsc.subcore_barrier()

    def run_type(x_ref, src_ref, dst_ref, sum_out):
        def body(i, carry):
            g = s + i * NS

            @pl.when(g < N_GROUPS)
            def _():
                pltpu.sync_copy(src_ref.at[pl.ds(g * GRP, GRP)], is8)
                pltpu.sync_copy(dst_ref.at[pl.ds(g * GRP, GRP)], id8)
                cp = pltpu.async_copy(x_ref.at[is8.at[0]], rows[0], sems[0])
                for j in range(GRP):
                    nxt = (j + 1) % 2
                    if j + 1 < GRP:
                        cp_n = pltpu.async_copy(x_ref.at[is8.at[j + 1]],
                                                rows[nxt], sems[nxt])
                    cp.wait()
                    pltpu.sync_copy(rows[j % 2], sum_acc.at[id8.at[j]], add=True)
                    if j + 1 < GRP:
                        cp = cp_n
            return carry
        lax.fori_loop(0, (N_GROUPS + NS - 1) // NS, body, 0)

        @pl.when(s == 0)
        def _():
            pltpu.sync_copy(src_ref.at[pl.ds(N_GROUPS * GRP, REM)],
                            is8.at[pl.ds(0, REM)])
            pltpu.sync_copy(dst_ref.at[pl.ds(N_GROUPS * GRP, REM)],
                            id8.at[pl.ds(0, REM)])
            for j in range(REM):
                pltpu.async_copy(x_ref.at[is8.at[j]], rows_a, sem_a).wait()
                pltpu.sync_copy(rows_a, sum_acc.at[id8.at[j]], add=True)

        plsc.subcore_barrier()
        pltpu.sync_copy(sum_acc.at[pl.ds(base, STRIPE)],
                        sum_out.at[pl.ds(base, STRIPE)])

        @pl.when(s == NS - 1)
        def _():
            pltpu.sync_copy(sum_acc.at[pl.ds(NS * STRIPE, 16)],
                            sum_out.at[pl.ds(NS * STRIPE, 16)])

    @pl.when(c == 0)
    def _():
        run_type(x_gene, src_g2d, dst_g2d, sum_dis)

    @pl.when(c == 1)
    def _():
        run_type(x_dis, src_d2g, dst_d2g, sum_gene)


def _make_agg():
    return pl.kernel(
        _agg_body,
        out_type=[
            jax.ShapeDtypeStruct((N, D), jnp.float32),
            jax.ShapeDtypeStruct((N, D), jnp.float32),
        ],
        mesh=_mesh(),
        scratch_types=[
            pltpu.VMEM((GRP, 128), jnp.int32),
            pltpu.VMEM((GRP, 128), jnp.int32),
            pltpu.VMEM((128, D), jnp.float32),
            pltpu.VMEM((128, D), jnp.float32),
            pltpu.VMEM_SHARED((N, D), jnp.float32),
            pltpu.SemaphoreType.DMA,
            pltpu.SemaphoreType.DMA,
        ],
    )


def _cnt_body(dst_g2d, dst_d2g, cnt_dis, cnt_gene,
              idx_dst_v, zcnt_v, ones_v, cnt_acc, sem):
    c = lax.axis_index("c")
    s = lax.axis_index("s")

    def fill(i, carry):
        for j in range(8):
            zcnt_v[i, pl.ds(j * 16, 16)] = jnp.zeros((16,), jnp.float32)
            ones_v[i, pl.ds(j * 16, 16)] = jnp.ones((16,), jnp.float32)
        return carry
    lax.fori_loop(0, 128, fill, 0)

    base = s * STRIPE
    off = 0
    for sz in (128, 128, 128, 128, 112):
        pltpu.sync_copy(zcnt_v.at[pl.ds(0, sz)], cnt_acc.at[pl.ds(base + off, sz)])
        off += sz

    @pl.when(s == NS - 1)
    def _():
        pltpu.sync_copy(zcnt_v.at[pl.ds(0, 16)], cnt_acc.at[pl.ds(NS * STRIPE, 16)])
    plsc.subcore_barrier()

    def run_type(dst_ref, cnt_out):
        def body(i, carry):
            g = s + i * NS

            @pl.when(g < N_GROUPS)
            def _():
                pltpu.sync_copy(dst_ref.at[pl.ds(g * GRP, GRP)], idx_dst_v)
                cps = [pltpu.async_copy(ones_v, cnt_acc.at[idx_dst_v.at[j]],
                                        sem, add=True) for j in range(GRP)]
                for cp in cps:
                    cp.wait()
            return carry
        lax.fori_loop(0, (N_GROUPS + NS - 1) // NS, body, 0)

        @pl.when(s == 0)
        def _():
            pltpu.sync_copy(dst_ref.at[pl.ds(N_GROUPS * GRP, REM)],
                            idx_dst_v.at[pl.ds(0, REM)])
            for j in range(REM):
                pltpu.sync_copy(ones_v, cnt_acc.at[idx_dst_v.at[j]], add=True)
        plsc.subcore_barrier()
        pltpu.sync_copy(cnt_acc.at[pl.ds(base, STRIPE)],
                        cnt_out.at[pl.ds(base, STRIPE)])

        @pl.when(s == NS - 1)
        def _():
            pltpu.sync_copy(cnt_acc.at[pl.ds(NS * STRIPE, 16)],
                            cnt_out.at[pl.ds(NS * STRIPE, 16)])

    @pl.when(c == 0)
    def _():
        run_type(dst_g2d, cnt_dis)

    @pl.when(c == 1)
    def _():
        run_type(dst_d2g, cnt_gene)


def _make_cnt():
    return pl.kernel(
        _cnt_body,
        out_type=[
            jax.ShapeDtypeStruct((N, 128), jnp.float32),
            jax.ShapeDtypeStruct((N, 128), jnp.float32),
        ],
        mesh=_mesh(),
        scratch_types=[
            pltpu.VMEM((GRP, 128), jnp.int32),
            pltpu.VMEM((128, 128), jnp.float32),
            pltpu.VMEM((128, 128), jnp.float32),
            pltpu.VMEM_SHARED((N, 128), jnp.float32),
            pltpu.SemaphoreType.DMA,
        ],
    )


def _dense_one(x_dst, ssum, cnt, Wd, bd, Ws, bs, Wu, bu, gam, bet, relu):
    cnt1 = cnt[:, 0:1]
    aggr = jnp.where(cnt1 > 0, ssum / jnp.maximum(cnt1, 1.0), 0.0)
    dst_msg = jnp.dot(x_dst, Wd, preferred_element_type=jnp.float32) + bd
    src_msg = jnp.dot(aggr, Ws, preferred_element_type=jnp.float32) + bs
    h = (jnp.dot(dst_msg, Wu[0:D, :], preferred_element_type=jnp.float32)
         + jnp.dot(src_msg, Wu[D:2 * D, :], preferred_element_type=jnp.float32)
         + bu)
    m = jnp.mean(h, axis=0, keepdims=True)
    hc = h - m
    v = jnp.mean(hc * hc, axis=0, keepdims=True)
    h = gam * hc / jnp.sqrt(v + 1e-5) + bet
    if relu:
        h = jnp.where(h >= 0, h, 0.01 * h)
    return h


def _dense_body(x_ref, s_ref, c_ref,
                wd, bd, ws, bs, wu, bu, gam, bet,
                h_out, *, relu):
    h_out[...] = _dense_one(x_ref[...], s_ref[...], c_ref[...],
                            wd[...], bd[...], ws[...], bs[...],
                            wu[...], bu[...], gam[...], bet[...], relu)


def _make_dense(relu):
    return pl.pallas_call(
        functools.partial(_dense_body, relu=relu),
        out_shape=jax.ShapeDtypeStruct((N, D), jnp.float32),
    )


DEC_CH = L_ROWS // NS


def _dec_body(hg, hd, a_g2d, b_g2d, a_d2g, b_d2g,
              part_g2d, part_d2g,
              ia_v, ib_v, ra0, ra1, rb0, rb1, o_v,
              sa0, sa1, sb0, sb1):
    c = lax.axis_index("c")
    s = lax.axis_index("s")
    ras = (ra0, ra1)
    rbs = (rb0, rb1)
    sas = (sa0, sa1)
    sbs = (sb0, sb1)

    def run(tabA, tabB, a_ref, b_ref, out_ref):
        r0 = s * DEC_CH
        pltpu.sync_copy(a_ref.at[pl.ds(r0, DEC_CH)], ia_v)
        pltpu.sync_copy(b_ref.at[pl.ds(r0, DEC_CH)], ib_v)
        cpa = pltpu.async_copy(tabA.at[ia_v.at[0]], ras[0], sas[0])
        cpb = pltpu.async_copy(tabB.at[ib_v.at[0]], rbs[0], sbs[0])
        for j in range(DEC_CH):
            cur = j % 2
            nxt = (j + 1) % 2
            if j + 1 < DEC_CH:
                cpa_n = pltpu.async_copy(tabA.at[ia_v.at[j + 1]], ras[nxt], sas[nxt])
                cpb_n = pltpu.async_copy(tabB.at[ib_v.at[j + 1]], rbs[nxt], sbs[nxt])
            cpa.wait()
            cpb.wait()
            ra_v = ras[cur]
            rb_v = rbs[cur]

            def pair(p, carry2):
                acc = ra_v[p, pl.ds(0, 16)] * rb_v[p, pl.ds(0, 16)]
                for k in range(1, 8):
                    acc = acc + ra_v[p, pl.ds(k * 16, 16)] * rb_v[p, pl.ds(k * 16, 16)]
                o_v[p, :] = acc
                return carry2
            lax.fori_loop(0, 128, pair, 0, unroll=2)
            pltpu.sync_copy(o_v, out_ref.at[pl.ds((r0 + j) * 128, 128)])
            if j + 1 < DEC_CH:
                cpa = cpa_n
                cpb = cpb_n

    @pl.when(c == 0)
    def _():
        run(hg, hd, a_g2d, b_g2d, part_g2d)

    @pl.when(c == 1)
    def _():
        run(hd, hg, a_d2g, b_d2g, part_d2g)


def _make_dec():
    return pl.kernel(
        _dec_body,
        out_type=[
            jax.ShapeDtypeStruct((L, 16), jnp.float32),
            jax.ShapeDtypeStruct((L, 16), jnp.float32),
        ],
        mesh=_mesh(),
        scratch_types=[
            pltpu.VMEM((DEC_CH, 128), jnp.int32),
            pltpu.VMEM((DEC_CH, 128), jnp.int32),
            pltpu.VMEM((128, D), jnp.float32),
            pltpu.VMEM((128, D), jnp.float32),
            pltpu.VMEM((128, D), jnp.float32),
            pltpu.VMEM((128, D), jnp.float32),
            pltpu.VMEM((128, 16), jnp.float32),
            pltpu.SemaphoreType.DMA,
            pltpu.SemaphoreType.DMA,
            pltpu.SemaphoreType.DMA,
            pltpu.SemaphoreType.DMA,
        ],
    )


def _finish_body(pg_ref, pd_ref, o0_ref, o1_ref):
    grp = lax.broadcasted_iota(jnp.int32, (16 * 128, 128), 0) // 16
    col = lax.broadcasted_iota(jnp.int32, (16 * 128, 128), 1)
    m = (grp == col).astype(jnp.float32)
    hi = jax.lax.Precision.HIGHEST
    o0_ref[...] = jnp.dot(pg_ref[...], m, precision=hi,
                          preferred_element_type=jnp.float32)
    o1_ref[...] = jnp.dot(pd_ref[...], m, precision=hi,
                          preferred_element_type=jnp.float32)


def _make_finish():
    return pl.pallas_call(
        _finish_body,
        out_shape=[
            jax.ShapeDtypeStruct((L_ROWS, 128), jnp.float32),
            jax.ShapeDtypeStruct((L_ROWS, 128), jnp.float32),
        ],
    )


def kernel(x_gene, x_disease, params, edge_index_g2d, edge_index_d2g,
           edge_label_index_g2d, edge_label_index_d2g):
    src_g2d = edge_index_g2d[0].astype(jnp.int32)
    dst_g2d = edge_index_g2d[1].astype(jnp.int32)
    src_d2g = edge_index_d2g[0].astype(jnp.int32)
    dst_d2g = edge_index_d2g[1].astype(jnp.int32)
    a_g2d = edge_label_index_g2d[0].astype(jnp.int32)
    b_g2d = edge_label_index_g2d[1].astype(jnp.int32)
    a_d2g = edge_label_index_d2g[0].astype(jnp.int32)
    b_d2g = edge_label_index_d2g[1].astype(jnp.int32)

    def wset(p):
        return (p['W_dst'], p['b_dst'].reshape(1, D), p['W_src'],
                p['b_src'].reshape(1, D), p['W_upd'], p['b_upd'].reshape(1, D))

    def bnset(p):
        return (p['gamma'].reshape(1, D), p['beta'].reshape(1, D))

    agg = _make_agg()
    dec = _make_dec()

    src_g2d_2 = src_g2d.reshape(E_ROWS, 128)
    dst_g2d_2 = dst_g2d.reshape(E_ROWS, 128)
    src_d2g_2 = src_d2g.reshape(E_ROWS, 128)
    dst_d2g_2 = dst_d2g.reshape(E_ROWS, 128)

    cnt_d, cnt_g = _make_cnt()(dst_g2d_2, dst_d2g_2)

    dense_relu = _make_dense(True)
    dense_lin = _make_dense(False)

    sum_d1, sum_g1 = agg(
        x_gene, x_disease, src_g2d_2, dst_g2d_2, src_d2g_2, dst_d2g_2)
    h1d = dense_relu(x_disease, sum_d1, cnt_d,
                     *wset(params['conv1_g2d']), *bnset(params['bn1_disease']))
    h1g = dense_relu(x_gene, sum_g1, cnt_g,
                     *wset(params['conv1_d2g']), *bnset(params['bn1_gene']))

    sum_d2, sum_g2 = agg(
        h1g, h1d, src_g2d_2, dst_g2d_2, src_d2g_2, dst_d2g_2)
    h2d = dense_lin(h1d, sum_d2, cnt_d,
                    *wset(params['conv2_g2d']), *bnset(params['bn2_disease']))
    h2g = dense_lin(h1g, sum_g2, cnt_g,
                    *wset(params['conv2_d2g']), *bnset(params['bn2_gene']))

    pg, pd = dec(h2g, h2d,
                 a_g2d.reshape(L_ROWS, 128), b_g2d.reshape(L_ROWS, 128),
                 a_d2g.reshape(L_ROWS, 128), b_d2g.reshape(L_ROWS, 128))
    o0, o1 = _make_finish()(pg.reshape(L_ROWS, 16 * 128), pd.reshape(L_ROWS, 16 * 128))
    return jnp.stack([o0.reshape(L), o1.reshape(L)], axis=0)


# speedup vs baseline: 7.1631x; 1.0008x over previous
import functools

import jax
import jax.numpy as jnp
from jax import lax
from jax.experimental import pallas as pl
from jax.experimental.pallas import tpu as pltpu
from jax.experimental.pallas import tpu_sc as plsc

N = 10000
D = 128
E = 320000
L = 65536

E_ROWS = E // 128
L_ROWS = L // 128
NS = 16
STRIPE = 624


def _mesh():
    return plsc.VectorSubcoreMesh(core_axis_name="c", subcore_axis_name="s")


GRP = 8
N_GROUPS = E_ROWS // GRP
REM = E_ROWS - N_GROUPS * GRP


def _agg_body(x_gene, x_dis, src_g2d, dst_g2d, src_d2g, dst_d2g,
              sum_dis, sum_gene,
              is8, id8, rows_a, rows_b, sum_acc, sem_a, sem_b, ssem_a, ssem_b):
    c = lax.axis_index("c")
    s = lax.axis_index("s")
    rows = (rows_a, rows_b)
    sems = (sem_a, sem_b)
    ssems = (ssem_a, ssem_b)

    def fill(i, carry):
        for j in range(8):
            rows_a[i, pl.ds(j * 16, 16)] = jnp.zeros((16,), jnp.float32)
        return carry
    lax.fori_loop(0, 128, fill, 0)

    base = s * STRIPE
    off = 0
    for sz in (128, 128, 128, 128, 112):
        pltpu.sync_copy(rows_a.at[pl.ds(0, sz)], sum_acc.at[pl.ds(base + off, sz)])
        off += sz

    @pl.when(s == NS - 1)
    def _():
        pltpu.sync_copy(rows_a.at[pl.ds(0, 16)], sum_acc.at[pl.ds(NS * STRIPE, 16)])
    plsc.subcore_barrier()

    def run_type(x_ref, src_ref, dst_ref, sum_out):
        def body(i, carry):
            g = s + i * NS

            @pl.when(g < N_GROUPS)
            def _():
                pltpu.sync_copy(src_ref.at[pl.ds(g * GRP, GRP)], is8)
                pltpu.sync_copy(dst_ref.at[pl.ds(g * GRP, GRP)], id8)
                cp = pltpu.async_copy(x_ref.at[is8.at[0]], rows[0], sems[0])
                sc = [None, None]
                for j in range(GRP):
                    nxt = (j + 1) % 2
                    if j + 1 < GRP:
                        if sc[nxt] is not None:
                            sc[nxt].wait()
                            sc[nxt] = None
                        cp_n = pltpu.async_copy(x_ref.at[is8.at[j + 1]],
                                                rows[nxt], sems[nxt])
                    cp.wait()
                    sc[j % 2] = pltpu.async_copy(rows[j % 2],
                                                 sum_acc.at[id8.at[j]],
                                                 ssems[j % 2], add=True)
                    if j + 1 < GRP:
                        cp = cp_n
                for k in range(2):
                    if sc[k] is not None:
                        sc[k].wait()
            return carry
        lax.fori_loop(0, (N_GROUPS + NS - 1) // NS, body, 0)

        @pl.when(s == 0)
        def _():
            pltpu.sync_copy(src_ref.at[pl.ds(N_GROUPS * GRP, REM)],
                            is8.at[pl.ds(0, REM)])
            pltpu.sync_copy(dst_ref.at[pl.ds(N_GROUPS * GRP, REM)],
                            id8.at[pl.ds(0, REM)])
            for j in range(REM):
                pltpu.async_copy(x_ref.at[is8.at[j]], rows_a, sem_a).wait()
                pltpu.sync_copy(rows_a, sum_acc.at[id8.at[j]], add=True)

        plsc.subcore_barrier()
        pltpu.sync_copy(sum_acc.at[pl.ds(base, STRIPE)],
                        sum_out.at[pl.ds(base, STRIPE)])

        @pl.when(s == NS - 1)
        def _():
            pltpu.sync_copy(sum_acc.at[pl.ds(NS * STRIPE, 16)],
                            sum_out.at[pl.ds(NS * STRIPE, 16)])

    @pl.when(c == 0)
    def _():
        run_type(x_gene, src_g2d, dst_g2d, sum_dis)

    @pl.when(c == 1)
    def _():
        run_type(x_dis, src_d2g, dst_d2g, sum_gene)


def _make_agg():
    return pl.kernel(
        _agg_body,
        out_type=[
            jax.ShapeDtypeStruct((N, D), jnp.float32),
            jax.ShapeDtypeStruct((N, D), jnp.float32),
        ],
        mesh=_mesh(),
        scratch_types=[
            pltpu.VMEM((GRP, 128), jnp.int32),
            pltpu.VMEM((GRP, 128), jnp.int32),
            pltpu.VMEM((128, D), jnp.float32),
            pltpu.VMEM((128, D), jnp.float32),
            pltpu.VMEM_SHARED((N, D), jnp.float32),
            pltpu.SemaphoreType.DMA,
            pltpu.SemaphoreType.DMA,
            pltpu.SemaphoreType.DMA,
            pltpu.SemaphoreType.DMA,
        ],
    )


def _cnt_body(dst_g2d, dst_d2g, cnt_dis, cnt_gene,
              idx_dst_v, zcnt_v, ones_v, cnt_acc, sem):
    c = lax.axis_index("c")
    s = lax.axis_index("s")

    def fill(i, carry):
        for j in range(8):
            zcnt_v[i, pl.ds(j * 16, 16)] = jnp.zeros((16,), jnp.float32)
            ones_v[i, pl.ds(j * 16, 16)] = jnp.ones((16,), jnp.float32)
        return carry
    lax.fori_loop(0, 128, fill, 0)

    base = s * STRIPE
    off = 0
    for sz in (128, 128, 128, 128, 112):
        pltpu.sync_copy(zcnt_v.at[pl.ds(0, sz)], cnt_acc.at[pl.ds(base + off, sz)])
        off += sz

    @pl.when(s == NS - 1)
    def _():
        pltpu.sync_copy(zcnt_v.at[pl.ds(0, 16)], cnt_acc.at[pl.ds(NS * STRIPE, 16)])
    plsc.subcore_barrier()

    def run_type(dst_ref, cnt_out):
        def body(i, carry):
            g = s + i * NS

            @pl.when(g < N_GROUPS)
            def _():
                pltpu.sync_copy(dst_ref.at[pl.ds(g * GRP, GRP)], idx_dst_v)
                cps = [pltpu.async_copy(ones_v, cnt_acc.at[idx_dst_v.at[j]],
                                        sem, add=True) for j in range(GRP)]
                for cp in cps:
                    cp.wait()
            return carry
        lax.fori_loop(0, (N_GROUPS + NS - 1) // NS, body, 0)

        @pl.when(s == 0)
        def _():
            pltpu.sync_copy(dst_ref.at[pl.ds(N_GROUPS * GRP, REM)],
                            idx_dst_v.at[pl.ds(0, REM)])
            for j in range(REM):
                pltpu.sync_copy(ones_v, cnt_acc.at[idx_dst_v.at[j]], add=True)
        plsc.subcore_barrier()
        pltpu.sync_copy(cnt_acc.at[pl.ds(base, STRIPE)],
                        cnt_out.at[pl.ds(base, STRIPE)])

        @pl.when(s == NS - 1)
        def _():
            pltpu.sync_copy(cnt_acc.at[pl.ds(NS * STRIPE, 16)],
                            cnt_out.at[pl.ds(NS * STRIPE, 16)])

    @pl.when(c == 0)
    def _():
        run_type(dst_g2d, cnt_dis)

    @pl.when(c == 1)
    def _():
        run_type(dst_d2g, cnt_gene)


def _make_cnt():
    return pl.kernel(
        _cnt_body,
        out_type=[
            jax.ShapeDtypeStruct((N, 128), jnp.float32),
            jax.ShapeDtypeStruct((N, 128), jnp.float32),
        ],
        mesh=_mesh(),
        scratch_types=[
            pltpu.VMEM((GRP, 128), jnp.int32),
            pltpu.VMEM((128, 128), jnp.float32),
            pltpu.VMEM((128, 128), jnp.float32),
            pltpu.VMEM_SHARED((N, 128), jnp.float32),
            pltpu.SemaphoreType.DMA,
        ],
    )


def _dense_one(x_dst, ssum, cnt, Wd, bd, Ws, bs, Wu, bu, gam, bet, relu):
    cnt1 = cnt[:, 0:1]
    aggr = jnp.where(cnt1 > 0, ssum / jnp.maximum(cnt1, 1.0), 0.0)
    dst_msg = jnp.dot(x_dst, Wd, preferred_element_type=jnp.float32) + bd
    src_msg = jnp.dot(aggr, Ws, preferred_element_type=jnp.float32) + bs
    h = (jnp.dot(dst_msg, Wu[0:D, :], preferred_element_type=jnp.float32)
         + jnp.dot(src_msg, Wu[D:2 * D, :], preferred_element_type=jnp.float32)
         + bu)
    m = jnp.mean(h, axis=0, keepdims=True)
    hc = h - m
    v = jnp.mean(hc * hc, axis=0, keepdims=True)
    h = gam * hc / jnp.sqrt(v + 1e-5) + bet
    if relu:
        h = jnp.where(h >= 0, h, 0.01 * h)
    return h


def _dense_body(x_ref, s_ref, c_ref,
                wd, bd, ws, bs, wu, bu, gam, bet,
                h_out, *, relu):
    h_out[...] = _dense_one(x_ref[...], s_ref[...], c_ref[...],
                            wd[...], bd[...], ws[...], bs[...],
                            wu[...], bu[...], gam[...], bet[...], relu)


def _make_dense(relu):
    return pl.pallas_call(
        functools.partial(_dense_body, relu=relu),
        out_shape=jax.ShapeDtypeStruct((N, D), jnp.float32),
    )


DEC_CH = L_ROWS // NS


def _dec_body(hg, hd, a_g2d, b_g2d, a_d2g, b_d2g,
              part_g2d, part_d2g,
              ia_v, ib_v, ra0, ra1, rb0, rb1, o_v,
              sa0, sa1, sb0, sb1):
    c = lax.axis_index("c")
    s = lax.axis_index("s")
    ras = (ra0, ra1)
    rbs = (rb0, rb1)
    sas = (sa0, sa1)
    sbs = (sb0, sb1)

    def run(tabA, tabB, a_ref, b_ref, out_ref):
        r0 = s * DEC_CH
        pltpu.sync_copy(a_ref.at[pl.ds(r0, DEC_CH)], ia_v)
        pltpu.sync_copy(b_ref.at[pl.ds(r0, DEC_CH)], ib_v)
        cpa = pltpu.async_copy(tabA.at[ia_v.at[0]], ras[0], sas[0])
        cpb = pltpu.async_copy(tabB.at[ib_v.at[0]], rbs[0], sbs[0])
        for j in range(DEC_CH):
            cur = j % 2
            nxt = (j + 1) % 2
            if j + 1 < DEC_CH:
                cpa_n = pltpu.async_copy(tabA.at[ia_v.at[j + 1]], ras[nxt], sas[nxt])
                cpb_n = pltpu.async_copy(tabB.at[ib_v.at[j + 1]], rbs[nxt], sbs[nxt])
            cpa.wait()
            cpb.wait()
            ra_v = ras[cur]
            rb_v = rbs[cur]

            def pair(p, carry2):
                acc = ra_v[p, pl.ds(0, 16)] * rb_v[p, pl.ds(0, 16)]
                for k in range(1, 8):
                    acc = acc + ra_v[p, pl.ds(k * 16, 16)] * rb_v[p, pl.ds(k * 16, 16)]
                o_v[p, :] = acc
                return carry2
            lax.fori_loop(0, 128, pair, 0, unroll=2)
            pltpu.sync_copy(o_v, out_ref.at[pl.ds((r0 + j) * 128, 128)])
            if j + 1 < DEC_CH:
                cpa = cpa_n
                cpb = cpb_n

    @pl.when(c == 0)
    def _():
        run(hg, hd, a_g2d, b_g2d, part_g2d)

    @pl.when(c == 1)
    def _():
        run(hd, hg, a_d2g, b_d2g, part_d2g)


def _make_dec():
    return pl.kernel(
        _dec_body,
        out_type=[
            jax.ShapeDtypeStruct((L, 16), jnp.float32),
            jax.ShapeDtypeStruct((L, 16), jnp.float32),
        ],
        mesh=_mesh(),
        scratch_types=[
            pltpu.VMEM((DEC_CH, 128), jnp.int32),
            pltpu.VMEM((DEC_CH, 128), jnp.int32),
            pltpu.VMEM((128, D), jnp.float32),
            pltpu.VMEM((128, D), jnp.float32),
            pltpu.VMEM((128, D), jnp.float32),
            pltpu.VMEM((128, D), jnp.float32),
            pltpu.VMEM((128, 16), jnp.float32),
            pltpu.SemaphoreType.DMA,
            pltpu.SemaphoreType.DMA,
            pltpu.SemaphoreType.DMA,
            pltpu.SemaphoreType.DMA,
        ],
    )


def _finish_body(pg_ref, pd_ref, o0_ref, o1_ref):
    grp = lax.broadcasted_iota(jnp.int32, (16 * 128, 128), 0) // 16
    col = lax.broadcasted_iota(jnp.int32, (16 * 128, 128), 1)
    m = (grp == col).astype(jnp.float32)
    hi = jax.lax.Precision.HIGHEST
    o0_ref[...] = jnp.dot(pg_ref[...], m, precision=hi,
                          preferred_element_type=jnp.float32)
    o1_ref[...] = jnp.dot(pd_ref[...], m, precision=hi,
                          preferred_element_type=jnp.float32)


def _make_finish():
    return pl.pallas_call(
        _finish_body,
        out_shape=[
            jax.ShapeDtypeStruct((L_ROWS, 128), jnp.float32),
            jax.ShapeDtypeStruct((L_ROWS, 128), jnp.float32),
        ],
    )


def kernel(x_gene, x_disease, params, edge_index_g2d, edge_index_d2g,
           edge_label_index_g2d, edge_label_index_d2g):
    src_g2d = edge_index_g2d[0].astype(jnp.int32)
    dst_g2d = edge_index_g2d[1].astype(jnp.int32)
    src_d2g = edge_index_d2g[0].astype(jnp.int32)
    dst_d2g = edge_index_d2g[1].astype(jnp.int32)
    a_g2d = edge_label_index_g2d[0].astype(jnp.int32)
    b_g2d = edge_label_index_g2d[1].astype(jnp.int32)
    a_d2g = edge_label_index_d2g[0].astype(jnp.int32)
    b_d2g = edge_label_index_d2g[1].astype(jnp.int32)

    def wset(p):
        return (p['W_dst'], p['b_dst'].reshape(1, D), p['W_src'],
                p['b_src'].reshape(1, D), p['W_upd'], p['b_upd'].reshape(1, D))

    def bnset(p):
        return (p['gamma'].reshape(1, D), p['beta'].reshape(1, D))

    agg = _make_agg()
    dec = _make_dec()

    src_g2d_2 = src_g2d.reshape(E_ROWS, 128)
    dst_g2d_2 = dst_g2d.reshape(E_ROWS, 128)
    src_d2g_2 = src_d2g.reshape(E_ROWS, 128)
    dst_d2g_2 = dst_d2g.reshape(E_ROWS, 128)

    cnt_d, cnt_g = _make_cnt()(dst_g2d_2, dst_d2g_2)

    dense_relu = _make_dense(True)
    dense_lin = _make_dense(False)

    sum_d1, sum_g1 = agg(
        x_gene, x_disease, src_g2d_2, dst_g2d_2, src_d2g_2, dst_d2g_2)
    h1d = dense_relu(x_disease, sum_d1, cnt_d,
                     *wset(params['conv1_g2d']), *bnset(params['bn1_disease']))
    h1g = dense_relu(x_gene, sum_g1, cnt_g,
                     *wset(params['conv1_d2g']), *bnset(params['bn1_gene']))

    sum_d2, sum_g2 = agg(
        h1g, h1d, src_g2d_2, dst_g2d_2, src_d2g_2, dst_d2g_2)
    h2d = dense_lin(h1d, sum_d2, cnt_d,
                    *wset(params['conv2_g2d']), *bnset(params['bn2_disease']))
    h2g = dense_lin(h1g, sum_g2, cnt_g,
                    *wset(params['conv2_d2g']), *bnset(params['bn2_gene']))

    pg, pd = dec(h2g, h2d,
                 a_g2d.reshape(L_ROWS, 128), b_g2d.reshape(L_ROWS, 128),
                 a_d2g.reshape(L_ROWS, 128), b_d2g.reshape(L_ROWS, 128))
    o0, o1 = _make_finish()(pg.reshape(L_ROWS, 16 * 128), pd.reshape(L_ROWS, 16 * 128))
    return jnp.stack([o0.reshape(L), o1.reshape(L)], axis=0)
